# baseline (speedup 1.0000x reference)
import functools

import jax
import jax.numpy as jnp
from jax import lax
from jax.experimental import pallas as pl
from jax.experimental.pallas import tpu as pltpu
from jax.experimental.pallas import tpu_sc as plsc

_ALPHA = 0.5


def _make_gather(C, D, B):
    NC, NS = 2, 16
    NW = NC * NS
    b_per_w = B // NW
    mesh = plsc.VectorSubcoreMesh(core_axis_name="c", subcore_axis_name="s")

    @functools.partial(
        pl.kernel,
        out_type=jax.ShapeDtypeStruct((B, D), jnp.float32),
        mesh=mesh,
        scratch_types=[
            pltpu.VMEM((b_per_w,), jnp.int32),
            pltpu.VMEM((b_per_w, D), jnp.float32),
            pltpu.SemaphoreType.DMA,
        ],
    )
    def gather_k(centers_hbm, idx_hbm, out_hbm, idx_v, rows_v, sem):
        wid = lax.axis_index("s") * NC + lax.axis_index("c")
        base = wid * b_per_w
        pltpu.sync_copy(idx_hbm.at[pl.ds(base, b_per_w)], idx_v)
        pltpu.async_copy(centers_hbm.at[idx_v], rows_v, sem).wait()
        pltpu.sync_copy(rows_v, out_hbm.at[pl.ds(base, b_per_w)])

    return gather_k


def _tc_body(lcol_ref, lrow_ref, f_blk_ref, cb_blk_ref, f_all_ref, cb_all_ref,
             res_ref, u_ref):
    lcol = lcol_ref[...]
    lrow = lrow_ref[...]
    eqf = (lcol == lrow).astype(jnp.float32)
    appear = jnp.sum(eqf, axis=1, keepdims=True)
    d_all = cb_all_ref[...] - f_all_ref[...]
    m = jax.lax.dot_general(
        eqf, d_all, (((1,), (0,)), ((), ())),
        preferred_element_type=jnp.float32)
    scale = _ALPHA / (1.0 + appear)
    cb_blk = cb_blk_ref[...]
    u_ref[...] = cb_blk - scale * m
    r = f_blk_ref[...] - cb_blk
    res_ref[...] = jnp.sum(r * r, axis=1, keepdims=True)


def _tc_math(labels, features, cb):
    B, D = features.shape
    BI = 512
    nblk = B // BI
    lcol = labels.reshape(B, 1)
    lrow = labels.reshape(1, B)
    return pl.pallas_call(
        _tc_body,
        grid=(nblk,),
        in_specs=[
            pl.BlockSpec((BI, 1), lambda i: (i, 0)),
            pl.BlockSpec((1, B), lambda i: (0, 0)),
            pl.BlockSpec((BI, D), lambda i: (i, 0)),
            pl.BlockSpec((BI, D), lambda i: (i, 0)),
            pl.BlockSpec((B, D), lambda i: (0, 0)),
            pl.BlockSpec((B, D), lambda i: (0, 0)),
        ],
        out_specs=[
            pl.BlockSpec((BI, 1), lambda i: (i, 0)),
            pl.BlockSpec((BI, D), lambda i: (i, 0)),
        ],
        out_shape=[
            jax.ShapeDtypeStruct((B, 1), jnp.float32),
            jax.ShapeDtypeStruct((B, D), jnp.float32),
        ],
    )(lcol, lrow, features, cb, features, cb)


def _make_copy_scatter(C, D, B):
    NS = 16
    chunk = (-(-C // NS) + 7) // 8 * 8
    tail = C - (NS - 1) * chunk
    assert 0 < tail <= chunk and tail % 8 == 0
    upd_per_w = B // NS
    CH = 128
    mesh = plsc.VectorSubcoreMesh(
        core_axis_name="c", subcore_axis_name="s", num_cores=1)

    @functools.partial(
        pl.kernel,
        out_type=jax.ShapeDtypeStruct((C, D), jnp.float32),
        mesh=mesh,
        scratch_types=[
            pltpu.VMEM((CH,), jnp.int32),
            pltpu.VMEM((CH, D), jnp.float32),
            pltpu.SemaphoreType.DMA,
        ],
    )
    def copy_scatter_k(centers_hbm, idx_hbm, u_hbm, out_hbm, idx_v, rows_v, sem):
        wid = lax.axis_index("s")
        base = wid * chunk

        @pl.when(wid < NS - 1)
        def _():
            pltpu.async_copy(
                centers_hbm.at[pl.ds(base, chunk)],
                out_hbm.at[pl.ds(base, chunk)],
                sem,
            ).wait()

        @pl.when(wid == NS - 1)
        def _():
            pltpu.async_copy(
                centers_hbm.at[pl.ds(base, tail)],
                out_hbm.at[pl.ds(base, tail)],
                sem,
            ).wait()
        plsc.subcore_barrier()
        for c in range(upd_per_w // CH):
            ub = wid * upd_per_w + c * CH
            pltpu.sync_copy(idx_hbm.at[pl.ds(ub, CH)], idx_v)
            pltpu.sync_copy(u_hbm.at[pl.ds(ub, CH)], rows_v)
            pltpu.sync_copy(rows_v, out_hbm.at[idx_v])

    return copy_scatter_k


def kernel(features, labels, centers):
    labels = labels.reshape(-1).astype(jnp.int32)
    features = features.astype(jnp.float32)
    B, D = features.shape
    C = centers.shape[0]

    cb = _make_gather(C, D, B)(centers, labels)
    result, u = _tc_math(labels, features, cb)
    new_centers = _make_copy_scatter(C, D, B)(centers, labels, u)
    return (result, new_centers)


# speedup vs baseline: 9.5553x; 9.5553x over previous
import functools

import jax
import jax.numpy as jnp
from jax import lax
from jax.experimental import pallas as pl
from jax.experimental.pallas import tpu as pltpu
from jax.experimental.pallas import tpu_sc as plsc

_ALPHA = 0.5


def _make_gather(C, D, B):
    NC, NS = 2, 16
    NW = NC * NS
    b_per_w = B // NW
    mesh = plsc.VectorSubcoreMesh(core_axis_name="c", subcore_axis_name="s")

    @functools.partial(
        pl.kernel,
        out_type=jax.ShapeDtypeStruct((B, D), jnp.float32),
        mesh=mesh,
        scratch_types=[
            pltpu.VMEM((b_per_w,), jnp.int32),
            pltpu.VMEM((b_per_w, D), jnp.float32),
            pltpu.SemaphoreType.DMA,
        ],
    )
    def gather_k(centers_hbm, idx_hbm, out_hbm, idx_v, rows_v, sem):
        wid = lax.axis_index("s") * NC + lax.axis_index("c")
        base = wid * b_per_w
        pltpu.sync_copy(idx_hbm.at[pl.ds(base, b_per_w)], idx_v)
        pltpu.async_copy(centers_hbm.at[idx_v], rows_v, sem).wait()
        pltpu.sync_copy(rows_v, out_hbm.at[pl.ds(base, b_per_w)])

    return gather_k


def _tc_body(H, lcol_ref, lrow_ref, f_blk_ref, cb_blk_ref, f_all_ref,
             cb_all_ref, sent_ref, res_ref, val0_ref, val1_ref):
    lcol = lcol_ref[...]
    lrow = lrow_ref[...]
    eqf = (lcol == lrow).astype(jnp.float32)
    appear = jnp.sum(eqf, axis=1, keepdims=True)
    d_all = cb_all_ref[...] - f_all_ref[...]
    m = jax.lax.dot_general(
        eqf, d_all, (((1,), (0,)), ((), ())),
        preferred_element_type=jnp.float32)
    scale = _ALPHA / (1.0 + appear)
    cb_blk = cb_blk_ref[...]
    u = cb_blk - scale * m
    r = f_blk_ref[...] - cb_blk
    res_ref[...] = jnp.sum(r * r, axis=1, keepdims=True)

    def sent_row(s, idx):
        mask = (lrow == s).astype(jnp.float32)
        n = jnp.sum(mask)
        v = jax.lax.dot_general(
            mask, d_all, (((1,), (0,)), ((), ())),
            preferred_element_type=jnp.float32)
        return sent_ref[idx, :].reshape(1, -1) - (_ALPHA / (1.0 + n)) * v

    row0 = sent_row(0, 0)
    rowh = sent_row(H, 1)
    in0 = lcol < H
    val0_ref[...] = jnp.where(in0, u, row0)
    val1_ref[...] = jnp.where(in0, rowh, u)


def _tc_math(labels, features, cb, sent_centers, H):
    B, D = features.shape
    BI = 512
    nblk = B // BI
    lcol = labels.reshape(B, 1)
    lrow = labels.reshape(1, B)
    return pl.pallas_call(
        functools.partial(_tc_body, H),
        grid=(nblk,),
        in_specs=[
            pl.BlockSpec((BI, 1), lambda i: (i, 0)),
            pl.BlockSpec((1, B), lambda i: (0, 0)),
            pl.BlockSpec((BI, D), lambda i: (i, 0)),
            pl.BlockSpec((BI, D), lambda i: (i, 0)),
            pl.BlockSpec((B, D), lambda i: (0, 0)),
            pl.BlockSpec((B, D), lambda i: (0, 0)),
            pl.BlockSpec((2, D), lambda i: (0, 0)),
        ],
        out_specs=[
            pl.BlockSpec((BI, 1), lambda i: (i, 0)),
            pl.BlockSpec((BI, D), lambda i: (i, 0)),
            pl.BlockSpec((BI, D), lambda i: (i, 0)),
        ],
        out_shape=[
            jax.ShapeDtypeStruct((B, 1), jnp.float32),
            jax.ShapeDtypeStruct((B, D), jnp.float32),
            jax.ShapeDtypeStruct((B, D), jnp.float32),
        ],
    )(lcol, lrow, features, cb, features, cb, sent_centers)


def _make_copy_scatter(C, D, B):
    NC, NS = 2, 16
    H = C // 2
    CC = 200
    nchunks = H // CC
    slots = -(-nchunks // NS)
    upd_per_w = B // NS
    CH = 128
    assert H % CC == 0 and CC % 8 == 0 and upd_per_w % CH == 0
    mesh = plsc.VectorSubcoreMesh(core_axis_name="c", subcore_axis_name="s")

    @functools.partial(
        pl.kernel,
        out_type=jax.ShapeDtypeStruct((C, D), jnp.float32),
        mesh=mesh,
        scratch_types=[
            pltpu.VMEM((CC, D), jnp.float32),
            pltpu.VMEM((CC, D), jnp.float32),
            pltpu.VMEM((CH,), jnp.int32),
            pltpu.VMEM((CH, D), jnp.float32),
            pltpu.SemaphoreType.DMA,
            pltpu.SemaphoreType.DMA,
            pltpu.SemaphoreType.DMA,
            pltpu.SemaphoreType.DMA,
            pltpu.SemaphoreType.DMA,
        ],
    )
    def copy_scatter_k(centers_hbm, idx0_hbm, idx1_hbm, val0_hbm, val1_hbm,
                       out_hbm, buf0, buf1, idx_v, rows_v,
                       isem0, isem1, osem0, osem1, ssem):
        cid = lax.axis_index("c")
        sid = lax.axis_index("s")
        core_base = cid * H
        bufs = (buf0, buf1)
        isems = (isem0, isem1)
        osems = (osem0, osem1)

        def chunk_base(j):
            return core_base + (sid + j * NS) * CC

        def start_in(j):
            @pl.when(sid + j * NS < nchunks)
            def _():
                pltpu.async_copy(
                    centers_hbm.at[pl.ds(chunk_base(j), CC)],
                    bufs[j % 2], isems[j % 2])

        def wait_in(j):
            @pl.when(sid + j * NS < nchunks)
            def _():
                pltpu.make_async_copy(
                    centers_hbm.at[pl.ds(chunk_base(j), CC)],
                    bufs[j % 2], isems[j % 2]).wait()

        def start_out(j):
            @pl.when(sid + j * NS < nchunks)
            def _():
                pltpu.async_copy(
                    bufs[j % 2],
                    out_hbm.at[pl.ds(chunk_base(j), CC)], osems[j % 2])

        def wait_out(j):
            @pl.when(sid + j * NS < nchunks)
            def _():
                pltpu.make_async_copy(
                    bufs[j % 2],
                    out_hbm.at[pl.ds(chunk_base(j), CC)], osems[j % 2]).wait()

        start_in(0)
        for j in range(slots):
            if j >= 1:
                wait_out(j - 1)
            if j + 1 < slots:
                start_in(j + 1)
            wait_in(j)
            start_out(j)
        wait_out(slots - 1)

        plsc.subcore_barrier()

        for c in range(upd_per_w // CH):
            ub = sid * upd_per_w + c * CH

            @pl.when(cid == 0)
            def _():
                pltpu.sync_copy(idx0_hbm.at[pl.ds(ub, CH)], idx_v)
                pltpu.sync_copy(val0_hbm.at[pl.ds(ub, CH)], rows_v)
                pltpu.async_copy(rows_v, out_hbm.at[idx_v], ssem).wait()

            @pl.when(cid == 1)
            def _():
                pltpu.sync_copy(idx1_hbm.at[pl.ds(ub, CH)], idx_v)
                pltpu.sync_copy(val1_hbm.at[pl.ds(ub, CH)], rows_v)
                pltpu.async_copy(rows_v, out_hbm.at[idx_v], ssem).wait()

    return copy_scatter_k


def kernel(features, labels, centers):
    labels = labels.reshape(-1).astype(jnp.int32)
    features = features.astype(jnp.float32)
    B, D = features.shape
    C = centers.shape[0]
    H = C // 2

    cb = _make_gather(C, D, B)(centers, labels)
    sent_centers = jnp.concatenate(
        [lax.slice(centers, (0, 0), (1, D)),
         lax.slice(centers, (H, 0), (H + 1, D))], axis=0)
    result, val0, val1 = _tc_math(labels, features, cb, sent_centers, H)
    idx0 = jnp.where(labels < H, labels, 0).astype(jnp.int32)
    idx1 = jnp.where(labels >= H, labels, H).astype(jnp.int32)
    new_centers = _make_copy_scatter(C, D, B)(
        centers, idx0, idx1, val0, val1)
    return (result, new_centers)
